# baseline (speedup 1.0000x reference)
import math

import jax
import jax.numpy as jnp
from jax.experimental import pallas as pl
from jax.experimental.pallas import tpu as pltpu


def _round_up(x, m):
    return ((x + m - 1) // m) * m


def _self_output_kernel(x_ref, w_ref, b_ref, res_ref, o_ref):
    x = x_ref[...].astype(jnp.bfloat16)
    w = w_ref[...].astype(jnp.bfloat16)
    y = jax.lax.dot_general(
        x, w, dimension_numbers=(((1,), (1,)), ((), ())),
        preferred_element_type=jnp.float32)
    o_ref[...] = y + b_ref[...] + res_ref[...]


def kernel(hidden_states, input_tensor, weight, bias):
    orig_shape = hidden_states.shape
    H = orig_shape[-1]
    M = math.prod(orig_shape[:-1])
    dtype = hidden_states.dtype

    H_pad = _round_up(H, 128)

    x2 = hidden_states.reshape(M, H)
    r2 = input_tensor.reshape(M, H)
    b2 = bias.reshape(1, H)
    if H_pad != H:
        x2 = jnp.pad(x2, ((0, 0), (0, H_pad - H)))
        r2 = jnp.pad(r2, ((0, 0), (0, H_pad - H)))
        weight = jnp.pad(weight, ((0, H_pad - H), (0, H_pad - H)))
        b2 = jnp.pad(b2, ((0, 0), (0, H_pad - H)))

    TM = min(2560, max(8, _round_up(M, 8)))

    grid = (pl.cdiv(M, TM),)
    cost = pl.CostEstimate(
        flops=2 * M * H_pad * H_pad,
        transcendentals=0,
        bytes_accessed=4 * (3 * M * H_pad + H_pad) + 4 * H_pad * H_pad,
    )
    out = pl.pallas_call(
        _self_output_kernel,
        out_shape=jax.ShapeDtypeStruct((M, H_pad), dtype),
        grid=grid,
        in_specs=[
            pl.BlockSpec((TM, H_pad), lambda i: (i, 0)),
            pl.BlockSpec((H_pad, H_pad), lambda i: (0, 0)),
            pl.BlockSpec((1, H_pad), lambda i: (0, 0)),
            pl.BlockSpec((TM, H_pad), lambda i: (i, 0)),
        ],
        out_specs=pl.BlockSpec((TM, H_pad), lambda i: (i, 0)),
        compiler_params=pltpu.CompilerParams(
            dimension_semantics=("parallel",),
            vmem_limit_bytes=60 << 20,
        ),
        cost_estimate=cost,
    )(x2, weight, b2, r2)

    if H_pad != H:
        out = out[:, :H]
    return out.reshape(orig_shape)


# speedup vs baseline: 1.0086x; 1.0086x over previous
import math

import jax
import jax.numpy as jnp
from jax.experimental import pallas as pl
from jax.experimental.pallas import tpu as pltpu


def _round_up(x, m):
    return ((x + m - 1) // m) * m


def _self_output_kernel(x_ref, w_ref, b_ref, res_ref, o_ref):
    x = x_ref[...].astype(jnp.bfloat16)
    w = w_ref[...].astype(jnp.bfloat16)
    y = jax.lax.dot_general(
        x, w, dimension_numbers=(((1,), (1,)), ((), ())),
        preferred_element_type=jnp.float32)
    o_ref[...] = y + b_ref[...] + res_ref[...]


def kernel(hidden_states, input_tensor, weight, bias):
    orig_shape = hidden_states.shape
    H = orig_shape[-1]
    M = math.prod(orig_shape[:-1])
    dtype = hidden_states.dtype

    H_pad = _round_up(H, 128)

    x2 = hidden_states.reshape(M, H)
    r2 = input_tensor.reshape(M, H)
    b2 = bias.reshape(1, H)
    if H_pad != H:
        x2 = jnp.pad(x2, ((0, 0), (0, H_pad - H)))
        r2 = jnp.pad(r2, ((0, 0), (0, H_pad - H)))
        weight = jnp.pad(weight, ((0, H_pad - H), (0, H_pad - H)))
        b2 = jnp.pad(b2, ((0, 0), (0, H_pad - H)))

    TM = min(2048, max(8, _round_up(M, 8)))

    grid = (pl.cdiv(M, TM),)
    cost = pl.CostEstimate(
        flops=2 * M * H_pad * H_pad,
        transcendentals=0,
        bytes_accessed=4 * (3 * M * H_pad + H_pad) + 4 * H_pad * H_pad,
    )
    out = pl.pallas_call(
        _self_output_kernel,
        out_shape=jax.ShapeDtypeStruct((M, H_pad), dtype),
        grid=grid,
        in_specs=[
            pl.BlockSpec((TM, H_pad), lambda i: (i, 0)),
            pl.BlockSpec((H_pad, H_pad), lambda i: (0, 0)),
            pl.BlockSpec((1, H_pad), lambda i: (0, 0)),
            pl.BlockSpec((TM, H_pad), lambda i: (i, 0)),
        ],
        out_specs=pl.BlockSpec((TM, H_pad), lambda i: (i, 0)),
        compiler_params=pltpu.CompilerParams(
            dimension_semantics=("parallel",),
            vmem_limit_bytes=60 << 20,
        ),
        cost_estimate=cost,
    )(x2, weight, b2, r2)

    if H_pad != H:
        out = out[:, :H]
    return out.reshape(orig_shape)
